# Initial kernel scaffold; baseline (speedup 1.0000x reference)
import jax
import jax.numpy as jnp
from jax.experimental import pallas as pl


def kernel(states, edge_ids, Wt, bt, gru_k0, gru_rk0, gru_b0, gru_k1, gru_rk1, gru_b1, gru_k2, gru_rk2, gru_b2, gru_k3, gru_rk3, gru_b3):
    raise NotImplementedError("write your pallas kernel here")


# speedup vs baseline: 6.2039x; 6.2039x over previous
import jax
import jax.numpy as jnp
from jax.experimental import pallas as pl

HID = 128
NT = 9
TS = [3, 1, 3, 1]
RES = {1: [0], 3: [0, 1]}


def _proj_body(x_ref, w_ref, b_ref, o_ref):
    o_ref[...] = (
        jnp.dot(x_ref[...], w_ref[...], preferred_element_type=jnp.float32)
        + b_ref[...]
    )


def _proj(h2, wall, bias, rb=2000):
    r = h2.shape[0]
    return pl.pallas_call(
        _proj_body,
        grid=(r // rb,),
        in_specs=[
            pl.BlockSpec((rb, HID), lambda i: (i, 0)),
            pl.BlockSpec((HID, NT * HID), lambda i: (0, 0)),
            pl.BlockSpec((1, NT * HID), lambda i: (0, 0)),
        ],
        out_specs=pl.BlockSpec((rb, NT * HID), lambda i: (i, 0)),
        out_shape=jax.ShapeDtypeStruct((r, NT * HID), jnp.float32),
    )(h2, wall, bias)


def _gru_body(x_ref, h_ref, k_ref, rk_ref, b0_ref, b1_ref, o_ref):
    mx = (
        jnp.dot(x_ref[...], k_ref[...], preferred_element_type=jnp.float32)
        + b0_ref[...]
    )
    mh = (
        jnp.dot(h_ref[...], rk_ref[...], preferred_element_type=jnp.float32)
        + b1_ref[...]
    )
    h = h_ref[...]
    z = jax.nn.sigmoid(mx[:, :HID] + mh[:, :HID])
    r = jax.nn.sigmoid(mx[:, HID:2 * HID] + mh[:, HID:2 * HID])
    hh = jnp.tanh(mx[:, 2 * HID:] + r * mh[:, 2 * HID:])
    o_ref[...] = z * h + (1.0 - z) * hh


def _gru(xcat, h, k, rk, b0, b1, rb=2000):
    r, d = xcat.shape
    return pl.pallas_call(
        _gru_body,
        grid=(r // rb,),
        in_specs=[
            pl.BlockSpec((rb, d), lambda i: (i, 0)),
            pl.BlockSpec((rb, HID), lambda i: (i, 0)),
            pl.BlockSpec((d, 3 * HID), lambda i: (0, 0)),
            pl.BlockSpec((HID, 3 * HID), lambda i: (0, 0)),
            pl.BlockSpec((1, 3 * HID), lambda i: (0, 0)),
            pl.BlockSpec((1, 3 * HID), lambda i: (0, 0)),
        ],
        out_specs=pl.BlockSpec((rb, HID), lambda i: (i, 0)),
        out_shape=jax.ShapeDtypeStruct((r, HID), jnp.float32),
    )(xcat, h, k, rk, b0, b1)


def kernel(states, edge_ids, Wt, bt, gru_k0, gru_rk0, gru_b0, gru_k1, gru_rk1,
           gru_b1, gru_k2, gru_rk2, gru_b2, gru_k3, gru_rk3, gru_b3):
    gk = [gru_k0, gru_k1, gru_k2, gru_k3]
    grk = [gru_rk0, gru_rk1, gru_rk2, gru_rk3]
    gb = [gru_b0, gru_b1, gru_b2, gru_b3]
    b, n, h_dim = states.shape
    bn = b * n
    etype = edge_ids[:, 0]
    eb = edge_ids[:, 1]
    es = edge_ids[:, 2]
    ed = edge_ids[:, 3]
    gidx = (eb * n + es) * NT + etype
    sidx = eb * n + ed
    layer_states = [states.reshape(bn, h_dim)]
    for l, steps in enumerate(TS):
        wall = Wt[l].transpose(1, 0, 2).reshape(h_dim, NT * h_dim)
        bias = bt[l].reshape(1, NT * h_dim)
        k, rk = gk[l], grk[l]
        b0, b1 = gb[l][0:1], gb[l][1:2]
        for s in range(steps):
            h = layer_states[-1]
            y = _proj(h, wall, bias)
            gathered = jnp.take(y.reshape(bn * NT, h_dim), gidx, axis=0)
            msgs = jnp.zeros((bn, h_dim), jnp.float32).at[sidx].add(gathered)
            parts = [layer_states[ix] for ix in RES.get(l, [])] + [msgs]
            xcat = jnp.concatenate(parts, axis=1) if len(parts) > 1 else msgs
            new = _gru(xcat, h, k, rk, b0, b1)
            if s == 0:
                layer_states.append(new)
            else:
                layer_states[-1] = new
    return layer_states[-1].reshape(b, n, h_dim)
